# baseline (speedup 1.0000x reference)
import jax
import jax.numpy as jnp
from jax.experimental import pallas as pl
from jax.experimental.pallas import tpu as pltpu

N = 10000
IN_F = 128
OUT_F = 32
TM = 200


def _gcn_kernel(x_ref, w_ref, b_ref, adj0_ref, adj1_ref, out_ref, sup_ref):
    @pl.when(pl.program_id(0) == 0)
    def _():
        sup_ref[...] = jnp.dot(x_ref[...], w_ref[...],
                               preferred_element_type=jnp.float32)

    acc0 = jnp.dot(adj0_ref[...], sup_ref[...],
                   preferred_element_type=jnp.float32)
    acc1 = jnp.dot(adj1_ref[...], sup_ref[...],
                   preferred_element_type=jnp.float32)
    acc = jnp.concatenate([acc0, acc1], axis=0)
    out_ref[...] = jax.nn.sigmoid(acc + b_ref[...])


@jax.jit
def kernel(input, adj, weight, bias):
    bias2d = bias.reshape(1, OUT_F)
    out = pl.pallas_call(
        _gcn_kernel,
        grid=(N // (2 * TM),),
        in_specs=[
            pl.BlockSpec((N, IN_F), lambda i: (0, 0)),
            pl.BlockSpec((IN_F, OUT_F), lambda i: (0, 0)),
            pl.BlockSpec((1, OUT_F), lambda i: (0, 0)),
            pl.BlockSpec((TM, N), lambda i: (2 * i, 0)),
            pl.BlockSpec((TM, N), lambda i: (2 * i + 1, 0)),
        ],
        out_specs=pl.BlockSpec((2 * TM, OUT_F), lambda i: (i, 0)),
        out_shape=jax.ShapeDtypeStruct((N, OUT_F), jnp.float32),
        scratch_shapes=[pltpu.VMEM((N, OUT_F), jnp.float32)],
        compiler_params=pltpu.CompilerParams(
            dimension_semantics=("arbitrary",),
        ),
    )(input, weight, bias2d, adj, adj)
    return out


# speedup vs baseline: 1.0009x; 1.0009x over previous
import jax
import jax.numpy as jnp
from jax.experimental import pallas as pl
from jax.experimental.pallas import tpu as pltpu

N = 10000
IN_F = 128
OUT_F = 32
TM = 200
NCHUNK = N // TM
NBUF = 4


def _gcn_kernel(x_ref, w_ref, b_ref, adj_ref, out_ref, sup_ref, buf_ref,
                sem_ref):
    def copy_in(chunk, slot):
        return pltpu.make_async_copy(
            adj_ref.at[pl.ds(chunk * TM, TM), :],
            buf_ref.at[slot],
            sem_ref.at[slot],
        )

    for s in range(NBUF):
        copy_in(s, s).start()

    sup_ref[...] = jnp.dot(x_ref[...], w_ref[...],
                           preferred_element_type=jnp.float32)

    def body(i, _):
        slot = jax.lax.rem(i, NBUF)
        copy_in(i, slot).wait()
        acc = jnp.dot(buf_ref[slot], sup_ref[...],
                      preferred_element_type=jnp.float32)
        out_ref[pl.ds(i * TM, TM), :] = jax.nn.sigmoid(acc + b_ref[...])

        @pl.when(i + NBUF < NCHUNK)
        def _():
            copy_in(i + NBUF, slot).start()

        return ()

    jax.lax.fori_loop(0, NCHUNK, body, (), unroll=False)


@jax.jit
def kernel(input, adj, weight, bias):
    bias2d = bias.reshape(1, OUT_F)
    out = pl.pallas_call(
        _gcn_kernel,
        in_specs=[
            pl.BlockSpec((N, IN_F), lambda: (0, 0)),
            pl.BlockSpec((IN_F, OUT_F), lambda: (0, 0)),
            pl.BlockSpec((1, OUT_F), lambda: (0, 0)),
            pl.BlockSpec(memory_space=pltpu.MemorySpace.HBM),
        ],
        out_specs=pl.BlockSpec((N, OUT_F), lambda: (0, 0)),
        out_shape=jax.ShapeDtypeStruct((N, OUT_F), jnp.float32),
        scratch_shapes=[
            pltpu.VMEM((N, OUT_F), jnp.float32),
            pltpu.VMEM((NBUF, TM, N), jnp.float32),
            pltpu.SemaphoreType.DMA((NBUF,)),
        ],
    )(input, weight, bias2d, adj)
    return out


# speedup vs baseline: 1.0089x; 1.0080x over previous
import jax
import jax.numpy as jnp
from jax.experimental import pallas as pl
from jax.experimental.pallas import tpu as pltpu

N = 10000
IN_F = 128
OUT_F = 32
TM = 200
NCHUNK = N // TM
NPAIR = NCHUNK // 2
NBUF = 2


def _gcn_kernel(x_ref, w_ref, b_ref, adj_ref, out_ref, sup_ref,
                buf0_ref, buf1_ref, sem0_ref, sem1_ref):
    def copy_even(pair, slot):
        return pltpu.make_async_copy(
            adj_ref.at[pl.ds(pair * 2 * TM, TM), :],
            buf0_ref.at[slot],
            sem0_ref.at[slot],
        )

    def copy_odd(pair, slot):
        return pltpu.make_async_copy(
            adj_ref.at[pl.ds((pair * 2 + 1) * TM, TM), :],
            buf1_ref.at[slot],
            sem1_ref.at[slot],
        )

    for s in range(NBUF):
        copy_even(s, s).start()
        copy_odd(s, s).start()

    sup_ref[...] = jnp.dot(x_ref[...], w_ref[...],
                           preferred_element_type=jnp.float32)

    def body(j, _):
        slot = jax.lax.rem(j, NBUF)
        copy_even(j, slot).wait()
        acc0 = jnp.dot(buf0_ref[slot], sup_ref[...],
                       preferred_element_type=jnp.float32)
        out_ref[pl.ds(j * 2 * TM, TM), :] = jax.nn.sigmoid(acc0 + b_ref[...])

        @pl.when(j + NBUF < NPAIR)
        def _():
            copy_even(j + NBUF, slot).start()

        copy_odd(j, slot).wait()
        acc1 = jnp.dot(buf1_ref[slot], sup_ref[...],
                       preferred_element_type=jnp.float32)
        out_ref[pl.ds((j * 2 + 1) * TM, TM), :] = jax.nn.sigmoid(
            acc1 + b_ref[...])

        @pl.when(j + NBUF < NPAIR)
        def _():
            copy_odd(j + NBUF, slot).start()

        return ()

    jax.lax.fori_loop(0, NPAIR, body, (), unroll=False)


@jax.jit
def kernel(input, adj, weight, bias):
    bias2d = bias.reshape(1, OUT_F)
    out = pl.pallas_call(
        _gcn_kernel,
        in_specs=[
            pl.BlockSpec((N, IN_F), lambda: (0, 0)),
            pl.BlockSpec((IN_F, OUT_F), lambda: (0, 0)),
            pl.BlockSpec((1, OUT_F), lambda: (0, 0)),
            pl.BlockSpec(memory_space=pltpu.MemorySpace.HBM),
        ],
        out_specs=pl.BlockSpec((N, OUT_F), lambda: (0, 0)),
        out_shape=jax.ShapeDtypeStruct((N, OUT_F), jnp.float32),
        scratch_shapes=[
            pltpu.VMEM((N, OUT_F), jnp.float32),
            pltpu.VMEM((NBUF, TM, N), jnp.float32),
            pltpu.VMEM((NBUF, TM, N), jnp.float32),
            pltpu.SemaphoreType.DMA((NBUF,)),
            pltpu.SemaphoreType.DMA((NBUF,)),
        ],
    )(input, weight, bias2d, adj)
    return out


# speedup vs baseline: 1.0119x; 1.0030x over previous
import jax
import jax.numpy as jnp
from jax.experimental import pallas as pl
from jax.experimental.pallas import tpu as pltpu

N = 10000
IN_F = 128
OUT_F = 32
TM = 400


def _gcn_kernel(adj_ref, x_ref, w_ref, b_ref, out_ref, sup_ref):
    i = pl.program_id(0)

    @pl.when(i == 0)
    def _():
        sup_ref[...] = jnp.dot(x_ref[...], w_ref[...],
                               preferred_element_type=jnp.float32)

    acc = jnp.dot(adj_ref[...], sup_ref[...],
                  preferred_element_type=jnp.float32)
    out_ref[pl.ds(i * TM, TM), :] = jax.nn.sigmoid(acc + b_ref[...])


@jax.jit
def kernel(input, adj, weight, bias):
    bias2d = bias.reshape(1, OUT_F)
    out = pl.pallas_call(
        _gcn_kernel,
        grid=(N // TM,),
        in_specs=[
            pl.BlockSpec((TM, N), lambda i: (i, 0)),
            pl.BlockSpec((N, IN_F), lambda i: (0, 0)),
            pl.BlockSpec((IN_F, OUT_F), lambda i: (0, 0)),
            pl.BlockSpec((1, OUT_F), lambda i: (0, 0)),
        ],
        out_specs=pl.BlockSpec((N, OUT_F), lambda i: (0, 0)),
        out_shape=jax.ShapeDtypeStruct((N, OUT_F), jnp.float32),
        scratch_shapes=[pltpu.VMEM((N, OUT_F), jnp.float32)],
        compiler_params=pltpu.CompilerParams(
            dimension_semantics=("arbitrary",),
        ),
    )(adj, input, weight, bias2d)
    return out


# speedup vs baseline: 1.0352x; 1.0230x over previous
import jax
import jax.numpy as jnp
from jax.experimental import pallas as pl
from jax.experimental.pallas import tpu as pltpu

N = 10000
IN_F = 128
OUT_F = 32
TM = 400


def _gcn_kernel(x_ref, w_ref, b_ref, adj_ref, out_ref, sup_ref):
    @pl.when(pl.program_id(0) == 0)
    def _():
        sup_ref[...] = jnp.dot(x_ref[...], w_ref[...],
                               preferred_element_type=jnp.float32)

    acc = jnp.dot(adj_ref[...], sup_ref[...],
                  preferred_element_type=jnp.float32)
    out_ref[...] = jax.nn.sigmoid(acc + b_ref[...])


@jax.jit
def kernel(input, adj, weight, bias):
    bias2d = bias.reshape(1, OUT_F)
    out = pl.pallas_call(
        _gcn_kernel,
        grid=(N // TM,),
        in_specs=[
            pl.BlockSpec((N, IN_F), lambda i: (0, 0)),
            pl.BlockSpec((IN_F, OUT_F), lambda i: (0, 0)),
            pl.BlockSpec((1, OUT_F), lambda i: (0, 0)),
            pl.BlockSpec((TM, N), lambda i: (i, 0)),
        ],
        out_specs=pl.BlockSpec((TM, OUT_F), lambda i: (i, 0)),
        out_shape=jax.ShapeDtypeStruct((N, OUT_F), jnp.float32),
        scratch_shapes=[pltpu.VMEM((N, OUT_F), jnp.float32)],
        compiler_params=pltpu.CompilerParams(
            dimension_semantics=("arbitrary",),
        ),
    )(input, weight, bias2d, adj)
    return out
